# Initial kernel scaffold; baseline (speedup 1.0000x reference)
import jax
import jax.numpy as jnp
from jax.experimental import pallas as pl


def kernel(effective_pressure, edge_index, link_length, geometric_gradient, discharge, sliding_velocity_link, overburden_pressure, status_at_node):
    raise NotImplementedError("write your pallas kernel here")


# speedup vs baseline: 60.3455x; 60.3455x over previous
import jax
import jax.numpy as jnp
from jax import lax
from jax.experimental import pallas as pl
from jax.experimental.pallas import tpu as pltpu
from jax.experimental.pallas import tpu_sc as plsc

_N_NODES = 100000
_NPAD = 100352
_N_EDGES = 1600000
_EPAD = 1638400
_ROWS = _EPAD // 128
_ROWS_PER_TILE = _ROWS // 32
_CROWS = 8
_CHUNKS = _ROWS_PER_TILE // _CROWS
_SLICE = _NPAD // 16
_QCH = _SLICE // 7
_SEC_PER_A = 31556926.0


def _ep_body(eff_ref, over_ref, stat_ref, ep_ref):
    ep_ref[...] = jnp.where(stat_ref[...] != 0, over_ref[...], eff_ref[...])


def _sc_body(ep_hbm, src_hbm, dst_hbm, len_hbm, sld_hbm, out_hbm,
             ep_v, src_v, dst_v, len_v, sld_v, grad_v, ones_v, stg_v,
             acc_g, acc_s, acc_c):
    cid = lax.axis_index("c")
    sid = lax.axis_index("s")
    wid = cid * 16 + sid

    for k in range(_QCH // 16):
        stg_v[pl.ds(k * 16, 16)] = jnp.zeros((16,), jnp.float32)
    for acc in (acc_g, acc_s, acc_c):
        for q in range(7):
            pltpu.sync_copy(stg_v, acc.at[pl.ds(sid * _SLICE + q * _QCH, _QCH)])
    for k in range(8):
        ones_v[pl.ds(k * 16, 16)] = jnp.full((16,), 1.0, jnp.float32)
    pltpu.sync_copy(ep_hbm, ep_v)
    plsc.subcore_barrier()

    row_base = wid * _ROWS_PER_TILE

    def chunk(c, carry):
        r0 = row_base + c * _CROWS
        pltpu.sync_copy(src_hbm.at[pl.ds(r0, _CROWS)], src_v)
        pltpu.sync_copy(dst_hbm.at[pl.ds(r0, _CROWS)], dst_v)
        pltpu.sync_copy(len_hbm.at[pl.ds(r0, _CROWS)], len_v)
        pltpu.sync_copy(sld_hbm.at[pl.ds(r0, _CROWS)], sld_v)
        for j in range(_CROWS):
            for k in range(8):
                sl = (j, pl.ds(k * 16, 16))
                eps = plsc.load_gather(ep_v, [src_v[sl]])
                epd = plsc.load_gather(ep_v, [dst_v[sl]])
                grad_v[sl] = (epd - eps) / len_v[sl]
            pltpu.sync_copy(grad_v.at[j], acc_g.at[src_v.at[j]], add=True)
            pltpu.sync_copy(grad_v.at[j], acc_g.at[dst_v.at[j]], add=True)
            pltpu.sync_copy(sld_v.at[j], acc_s.at[src_v.at[j]], add=True)
            pltpu.sync_copy(sld_v.at[j], acc_s.at[dst_v.at[j]], add=True)
            pltpu.sync_copy(ones_v, acc_c.at[src_v.at[j]], add=True)
            pltpu.sync_copy(ones_v, acc_c.at[dst_v.at[j]], add=True)
        return carry

    lax.fori_loop(0, _CHUNKS, chunk, 0)
    plsc.subcore_barrier()

    for a, acc in enumerate((acc_g, acc_s, acc_c)):
        for q in range(7):
            off = sid * _SLICE + q * _QCH
            pltpu.sync_copy(acc.at[pl.ds(off, _QCH)], stg_v)
            pltpu.sync_copy(
                stg_v, out_hbm.at[pl.ds((cid * 3 + a) * _NPAD + off, _QCH)])


def _combine_body(p_ref, ep_ref, geo_ref, dis_ref, out_ref):
    opening_coeff = 1.3455e-09
    closure_coeff = 7.11e-24
    gsum = p_ref[0, 0] + p_ref[1, 0]
    ssum = p_ref[0, 1] + p_ref[1, 1]
    cnt = jnp.maximum(p_ref[0, 2] + p_ref[1, 2], 1.0)
    ep = ep_ref[...]
    dis = dis_ref[...]
    gradient = gsum / cnt + geo_ref[...]
    cavity = jnp.abs(ssum / cnt / _SEC_PER_A) * 0.03
    conduit = (opening_coeff * dis * gradient + cavity) / (
        cavity / 5.74 + closure_coeff * ep * ep * ep)
    conduit = jnp.where(conduit < 1e-06, 1e-06, conduit)
    out_ref[...] = dis - (opening_coeff * conduit ** 1.25
                          * (jnp.abs(gradient) + 1e-12) ** (-0.5) * gradient)


def kernel(effective_pressure, edge_index, link_length, geometric_gradient,
           discharge, sliding_velocity_link, overburden_pressure, status_at_node):
    f32 = jnp.float32
    npad = _NPAD - _N_NODES
    eff = jnp.pad(effective_pressure, (0, npad), constant_values=1.0)
    over = jnp.pad(overburden_pressure, (0, npad), constant_values=1.0)
    stat = jnp.pad(status_at_node, (0, npad))
    geo = jnp.pad(geometric_gradient, (0, npad))
    dis = jnp.pad(discharge, (0, npad))
    epad = _EPAD - _N_EDGES
    src = jnp.pad(edge_index[0], (0, epad), constant_values=_N_NODES)
    dst = jnp.pad(edge_index[1], (0, epad), constant_values=_N_NODES)
    ln = jnp.pad(link_length, (0, epad), constant_values=1.0)
    sld = jnp.pad(sliding_velocity_link, (0, epad))

    ep = pl.pallas_call(
        _ep_body,
        out_shape=jax.ShapeDtypeStruct((784, 128), f32),
    )(eff.reshape(784, 128), over.reshape(784, 128), stat.reshape(784, 128))

    sc = pl.kernel(
        _sc_body,
        out_type=jax.ShapeDtypeStruct((6 * _NPAD,), f32),
        mesh=plsc.VectorSubcoreMesh(core_axis_name="c", subcore_axis_name="s"),
        compiler_params=pltpu.CompilerParams(needs_layout_passes=False),
        scratch_types=[
            pltpu.VMEM((_NPAD,), f32),
            pltpu.VMEM((_CROWS, 128), jnp.int32),
            pltpu.VMEM((_CROWS, 128), jnp.int32),
            pltpu.VMEM((_CROWS, 128), f32),
            pltpu.VMEM((_CROWS, 128), f32),
            pltpu.VMEM((_CROWS, 128), f32),
            pltpu.VMEM((128,), f32),
            pltpu.VMEM((_QCH,), f32),
            pltpu.VMEM_SHARED((_NPAD,), f32),
            pltpu.VMEM_SHARED((_NPAD,), f32),
            pltpu.VMEM_SHARED((_NPAD,), f32),
        ],
    )
    partials = sc(ep.reshape(_NPAD), src.reshape(_ROWS, 128),
                  dst.reshape(_ROWS, 128), ln.reshape(_ROWS, 128),
                  sld.reshape(_ROWS, 128))

    residual = pl.pallas_call(
        _combine_body,
        out_shape=jax.ShapeDtypeStruct((784, 128), f32),
    )(partials.reshape(2, 3, 784, 128), ep,
      geo.reshape(784, 128), dis.reshape(784, 128))
    return residual.reshape(_NPAD)[:_N_NODES]


# speedup vs baseline: 62.5181x; 1.0360x over previous
import jax
import jax.numpy as jnp
from jax import lax
from jax.experimental import pallas as pl
from jax.experimental.pallas import tpu as pltpu
from jax.experimental.pallas import tpu_sc as plsc

_N_NODES = 100000
_NPAD = 100352
_N_EDGES = 1600000
_EPAD = 1638400
_EPT = _EPAD // 32
_CE = 1024
_CHUNKS = _EPT // _CE
_SLICE = _NPAD // 16
_QCH = _SLICE // 7
_SEC_PER_A = 31556926.0


def _ep_body(eff_ref, over_ref, stat_ref, ep_ref):
    ep_ref[...] = jnp.where(stat_ref[...] != 0, over_ref[...], eff_ref[...])


def _sc_body(ep_hbm, src_hbm, dst_hbm, len_hbm, sld_hbm, out_hbm,
             ep_v, src_v, dst_v, len_v, sld_v, grad_v, ones_v, stg_v,
             acc_g, acc_s, acc_c, sem):
    cid = lax.axis_index("c")
    sid = lax.axis_index("s")
    wid = cid * 16 + sid

    for k in range(_QCH // 16):
        stg_v[pl.ds(k * 16, 16)] = jnp.zeros((16,), jnp.float32)
    for acc in (acc_g, acc_s, acc_c):
        for q in range(7):
            pltpu.sync_copy(stg_v, acc.at[pl.ds(sid * _SLICE + q * _QCH, _QCH)])
    for k in range(_CE // 16):
        ones_v[pl.ds(k * 16, 16)] = jnp.full((16,), 1.0, jnp.float32)
    pltpu.sync_copy(ep_hbm, ep_v)
    plsc.subcore_barrier()

    edge_base = wid * _EPT

    def chunk(c, carry):
        e0 = edge_base + c * _CE
        pltpu.sync_copy(src_hbm.at[pl.ds(e0, _CE)], src_v)
        pltpu.sync_copy(dst_hbm.at[pl.ds(e0, _CE)], dst_v)
        pltpu.sync_copy(len_hbm.at[pl.ds(e0, _CE)], len_v)
        pltpu.sync_copy(sld_hbm.at[pl.ds(e0, _CE)], sld_v)
        for k in range(_CE // 16):
            sl = pl.ds(k * 16, 16)
            eps = plsc.load_gather(ep_v, [src_v[sl]])
            epd = plsc.load_gather(ep_v, [dst_v[sl]])
            grad_v[sl] = (epd - eps) / len_v[sl]
        descs = [
            pltpu.async_copy(grad_v, acc_g.at[src_v], sem, add=True),
            pltpu.async_copy(grad_v, acc_g.at[dst_v], sem, add=True),
            pltpu.async_copy(sld_v, acc_s.at[src_v], sem, add=True),
            pltpu.async_copy(sld_v, acc_s.at[dst_v], sem, add=True),
            pltpu.async_copy(ones_v, acc_c.at[src_v], sem, add=True),
            pltpu.async_copy(ones_v, acc_c.at[dst_v], sem, add=True),
        ]
        for d in descs:
            d.wait()
        return carry

    lax.fori_loop(0, _CHUNKS, chunk, 0)
    plsc.subcore_barrier()

    for a, acc in enumerate((acc_g, acc_s, acc_c)):
        for q in range(7):
            off = sid * _SLICE + q * _QCH
            pltpu.sync_copy(acc.at[pl.ds(off, _QCH)], stg_v)
            pltpu.sync_copy(
                stg_v, out_hbm.at[pl.ds((cid * 3 + a) * _NPAD + off, _QCH)])


def _combine_body(p_ref, ep_ref, geo_ref, dis_ref, out_ref):
    opening_coeff = 1.3455e-09
    closure_coeff = 7.11e-24
    gsum = p_ref[0, 0] + p_ref[1, 0]
    ssum = p_ref[0, 1] + p_ref[1, 1]
    cnt = jnp.maximum(p_ref[0, 2] + p_ref[1, 2], 1.0)
    ep = ep_ref[...]
    dis = dis_ref[...]
    gradient = gsum / cnt + geo_ref[...]
    cavity = jnp.abs(ssum / cnt / _SEC_PER_A) * 0.03
    conduit = (opening_coeff * dis * gradient + cavity) / (
        cavity / 5.74 + closure_coeff * ep * ep * ep)
    conduit = jnp.where(conduit < 1e-06, 1e-06, conduit)
    out_ref[...] = dis - (opening_coeff * conduit ** 1.25
                          * (jnp.abs(gradient) + 1e-12) ** (-0.5) * gradient)


def kernel(effective_pressure, edge_index, link_length, geometric_gradient,
           discharge, sliding_velocity_link, overburden_pressure, status_at_node):
    f32 = jnp.float32
    npad = _NPAD - _N_NODES
    eff = jnp.pad(effective_pressure, (0, npad), constant_values=1.0)
    over = jnp.pad(overburden_pressure, (0, npad), constant_values=1.0)
    stat = jnp.pad(status_at_node, (0, npad))
    geo = jnp.pad(geometric_gradient, (0, npad))
    dis = jnp.pad(discharge, (0, npad))
    epad = _EPAD - _N_EDGES
    src = jnp.pad(edge_index[0], (0, epad), constant_values=_N_NODES)
    dst = jnp.pad(edge_index[1], (0, epad), constant_values=_N_NODES)
    ln = jnp.pad(link_length, (0, epad), constant_values=1.0)
    sld = jnp.pad(sliding_velocity_link, (0, epad))

    ep = pl.pallas_call(
        _ep_body,
        out_shape=jax.ShapeDtypeStruct((784, 128), f32),
    )(eff.reshape(784, 128), over.reshape(784, 128), stat.reshape(784, 128))

    sc = pl.kernel(
        _sc_body,
        out_type=jax.ShapeDtypeStruct((6 * _NPAD,), f32),
        mesh=plsc.VectorSubcoreMesh(core_axis_name="c", subcore_axis_name="s"),
        compiler_params=pltpu.CompilerParams(needs_layout_passes=False),
        scratch_types=[
            pltpu.VMEM((_NPAD,), f32),
            pltpu.VMEM((_CE,), jnp.int32),
            pltpu.VMEM((_CE,), jnp.int32),
            pltpu.VMEM((_CE,), f32),
            pltpu.VMEM((_CE,), f32),
            pltpu.VMEM((_CE,), f32),
            pltpu.VMEM((_CE,), f32),
            pltpu.VMEM((_QCH,), f32),
            pltpu.VMEM_SHARED((_NPAD,), f32),
            pltpu.VMEM_SHARED((_NPAD,), f32),
            pltpu.VMEM_SHARED((_NPAD,), f32),
            pltpu.SemaphoreType.DMA,
        ],
    )
    partials = sc(ep.reshape(_NPAD), src, dst, ln, sld)

    residual = pl.pallas_call(
        _combine_body,
        out_shape=jax.ShapeDtypeStruct((784, 128), f32),
    )(partials.reshape(2, 3, 784, 128), ep,
      geo.reshape(784, 128), dis.reshape(784, 128))
    return residual.reshape(_NPAD)[:_N_NODES]


# speedup vs baseline: 73.4253x; 1.1745x over previous
import jax
import jax.numpy as jnp
from jax import lax
from jax.experimental import pallas as pl
from jax.experimental.pallas import tpu as pltpu
from jax.experimental.pallas import tpu_sc as plsc

_N_NODES = 100000
_NPAD = 100352
_N_EDGES = 1600000
_EPAD = 1638400
_EPT = _EPAD // 32
_CE = 1024
_CHUNKS = _EPT // _CE
_SLICE = _NPAD // 16
_QCH = _SLICE // 7
_SEC_PER_A = 31556926.0


def _ep_body(eff_ref, over_ref, stat_ref, ep_ref):
    ep_ref[...] = jnp.where(stat_ref[...] != 0, over_ref[...], eff_ref[...])


def _sc_body(ep_hbm, src_hbm, dst_hbm, len_hbm, sld_hbm, out_hbm,
             ep_v, src0, dst0, lg0, sld0, src1, dst1, lg1, sld1,
             ones_v, stg_v, acc_g, acc_s, acc_c,
             semL0, semL1, semS0, semS1):
    cid = lax.axis_index("c")
    sid = lax.axis_index("s")
    wid = cid * 16 + sid

    for k in range(_QCH // 16):
        stg_v[pl.ds(k * 16, 16)] = jnp.zeros((16,), jnp.float32)
    for acc in (acc_g, acc_s, acc_c):
        for q in range(7):
            pltpu.sync_copy(stg_v, acc.at[pl.ds(sid * _SLICE + q * _QCH, _QCH)])
    for k in range(_CE // 16):
        ones_v[pl.ds(k * 16, 16)] = jnp.full((16,), 1.0, jnp.float32)
    pltpu.sync_copy(ep_hbm, ep_v)

    edge_base = wid * _EPT

    def fire_loads(c, bufs, sem):
        e0 = edge_base + c * _CE
        pltpu.async_copy(src_hbm.at[pl.ds(e0, _CE)], bufs[0], sem)
        pltpu.async_copy(dst_hbm.at[pl.ds(e0, _CE)], bufs[1], sem)
        pltpu.async_copy(len_hbm.at[pl.ds(e0, _CE)], bufs[2], sem)
        pltpu.async_copy(sld_hbm.at[pl.ds(e0, _CE)], bufs[3], sem)

    def wait_loads(c, bufs, sem):
        e0 = edge_base + c * _CE
        pltpu.make_async_copy(src_hbm.at[pl.ds(e0, _CE)], bufs[0], sem).wait()
        pltpu.make_async_copy(dst_hbm.at[pl.ds(e0, _CE)], bufs[1], sem).wait()
        pltpu.make_async_copy(len_hbm.at[pl.ds(e0, _CE)], bufs[2], sem).wait()
        pltpu.make_async_copy(sld_hbm.at[pl.ds(e0, _CE)], bufs[3], sem).wait()

    def compute(bufs):
        src_v, dst_v, lg_v, _ = bufs
        for k in range(_CE // 16):
            sl = pl.ds(k * 16, 16)
            eps = plsc.load_gather(ep_v, [src_v[sl]])
            epd = plsc.load_gather(ep_v, [dst_v[sl]])
            lg_v[sl] = (epd - eps) / lg_v[sl]

    def scatter_descs(bufs, sem):
        src_v, dst_v, lg_v, sld_v = bufs
        return [
            pltpu.make_async_copy(lg_v, acc_g.at[src_v], sem),
            pltpu.make_async_copy(lg_v, acc_g.at[dst_v], sem),
            pltpu.make_async_copy(sld_v, acc_s.at[src_v], sem),
            pltpu.make_async_copy(sld_v, acc_s.at[dst_v], sem),
            pltpu.make_async_copy(ones_v, acc_c.at[src_v], sem),
            pltpu.make_async_copy(ones_v, acc_c.at[dst_v], sem),
        ]

    def fire_scatters(bufs, sem):
        for d in scatter_descs(bufs, sem):
            d.start(add=True)

    def wait_scatters(bufs, sem):
        for d in scatter_descs(bufs, sem):
            d.wait()

    bufs0 = (src0, dst0, lg0, sld0)
    bufs1 = (src1, dst1, lg1, sld1)

    fire_loads(0, bufs0, semL0)
    plsc.subcore_barrier()

    def pair(i, carry):
        c0 = 2 * i
        wait_loads(c0, bufs0, semL0)
        compute(bufs0)

        @pl.when(i > 0)
        def _():
            wait_scatters(bufs1, semS1)

        fire_loads(c0 + 1, bufs1, semL1)
        fire_scatters(bufs0, semS0)
        wait_loads(c0 + 1, bufs1, semL1)
        compute(bufs1)
        wait_scatters(bufs0, semS0)

        @pl.when(i < _CHUNKS // 2 - 1)
        def _():
            fire_loads(c0 + 2, bufs0, semL0)

        fire_scatters(bufs1, semS1)
        return carry

    lax.fori_loop(0, _CHUNKS // 2, pair, 0)
    wait_scatters(bufs1, semS1)
    plsc.subcore_barrier()

    for a, acc in enumerate((acc_g, acc_s, acc_c)):
        for q in range(7):
            off = sid * _SLICE + q * _QCH
            pltpu.sync_copy(acc.at[pl.ds(off, _QCH)], stg_v)
            pltpu.sync_copy(
                stg_v, out_hbm.at[pl.ds((cid * 3 + a) * _NPAD + off, _QCH)])


def _combine_body(p_ref, ep_ref, geo_ref, dis_ref, out_ref):
    opening_coeff = 1.3455e-09
    closure_coeff = 7.11e-24
    gsum = p_ref[0, 0] + p_ref[1, 0]
    ssum = p_ref[0, 1] + p_ref[1, 1]
    cnt = jnp.maximum(p_ref[0, 2] + p_ref[1, 2], 1.0)
    ep = ep_ref[...]
    dis = dis_ref[...]
    gradient = gsum / cnt + geo_ref[...]
    cavity = jnp.abs(ssum / cnt / _SEC_PER_A) * 0.03
    conduit = (opening_coeff * dis * gradient + cavity) / (
        cavity / 5.74 + closure_coeff * ep * ep * ep)
    conduit = jnp.where(conduit < 1e-06, 1e-06, conduit)
    out_ref[...] = dis - (opening_coeff * conduit ** 1.25
                          * (jnp.abs(gradient) + 1e-12) ** (-0.5) * gradient)


def kernel(effective_pressure, edge_index, link_length, geometric_gradient,
           discharge, sliding_velocity_link, overburden_pressure, status_at_node):
    f32 = jnp.float32
    npad = _NPAD - _N_NODES
    eff = jnp.pad(effective_pressure, (0, npad), constant_values=1.0)
    over = jnp.pad(overburden_pressure, (0, npad), constant_values=1.0)
    stat = jnp.pad(status_at_node, (0, npad))
    geo = jnp.pad(geometric_gradient, (0, npad))
    dis = jnp.pad(discharge, (0, npad))
    epad = _EPAD - _N_EDGES
    src = jnp.pad(edge_index[0], (0, epad), constant_values=_N_NODES)
    dst = jnp.pad(edge_index[1], (0, epad), constant_values=_N_NODES)
    ln = jnp.pad(link_length, (0, epad), constant_values=1.0)
    sld = jnp.pad(sliding_velocity_link, (0, epad))

    ep = pl.pallas_call(
        _ep_body,
        out_shape=jax.ShapeDtypeStruct((784, 128), f32),
    )(eff.reshape(784, 128), over.reshape(784, 128), stat.reshape(784, 128))

    sc = pl.kernel(
        _sc_body,
        out_type=jax.ShapeDtypeStruct((6 * _NPAD,), f32),
        mesh=plsc.VectorSubcoreMesh(core_axis_name="c", subcore_axis_name="s"),
        compiler_params=pltpu.CompilerParams(needs_layout_passes=False),
        scratch_types=[
            pltpu.VMEM((_NPAD,), f32),
            pltpu.VMEM((_CE,), jnp.int32),
            pltpu.VMEM((_CE,), jnp.int32),
            pltpu.VMEM((_CE,), f32),
            pltpu.VMEM((_CE,), f32),
            pltpu.VMEM((_CE,), jnp.int32),
            pltpu.VMEM((_CE,), jnp.int32),
            pltpu.VMEM((_CE,), f32),
            pltpu.VMEM((_CE,), f32),
            pltpu.VMEM((_CE,), f32),
            pltpu.VMEM((_QCH,), f32),
            pltpu.VMEM_SHARED((_NPAD,), f32),
            pltpu.VMEM_SHARED((_NPAD,), f32),
            pltpu.VMEM_SHARED((_NPAD,), f32),
            pltpu.SemaphoreType.DMA,
            pltpu.SemaphoreType.DMA,
            pltpu.SemaphoreType.DMA,
            pltpu.SemaphoreType.DMA,
        ],
    )
    partials = sc(ep.reshape(_NPAD), src, dst, ln, sld)

    residual = pl.pallas_call(
        _combine_body,
        out_shape=jax.ShapeDtypeStruct((784, 128), f32),
    )(partials.reshape(2, 3, 784, 128), ep,
      geo.reshape(784, 128), dis.reshape(784, 128))
    return residual.reshape(_NPAD)[:_N_NODES]


# speedup vs baseline: 151.2326x; 2.0597x over previous
import jax
import jax.numpy as jnp
from jax import lax
from jax.experimental import pallas as pl
from jax.experimental.pallas import tpu as pltpu
from jax.experimental.pallas import tpu_sc as plsc

_N_NODES = 100000
_NPAD = 100352
_N_EDGES = 1600000
_EPT = _N_EDGES // 32
_CE = 1024
_FULL = _EPT // _CE
_TAIL = _EPT - _FULL * _CE
_SLICE = _NPAD // 16
_QCH = _SLICE // 7
_SEC_PER_A = 31556926.0


def _ep_body(eff_ref, over_ref, stat_ref, ep_ref):
    ep_ref[...] = jnp.where(stat_ref[...] != 0, over_ref[...], eff_ref[...])


def _sc_body(ep_hbm, src_hbm, dst_hbm, len_hbm, sld_hbm, out_hbm,
             ep_v, src0, dst0, lg0, sld0, src1, dst1, lg1, sld1,
             ones_v, stg_v, acc_g, acc_s, acc_c,
             semL0, semL1, semS0, semS1):
    cid = lax.axis_index("c")
    sid = lax.axis_index("s")
    wid = cid * 16 + sid

    for k in range(_QCH // 16):
        stg_v[pl.ds(k * 16, 16)] = jnp.zeros((16,), jnp.float32)
    for acc in (acc_g, acc_s, acc_c):
        for q in range(7):
            pltpu.sync_copy(stg_v, acc.at[pl.ds(sid * _SLICE + q * _QCH, _QCH)])
    for k in range(_CE // 16):
        ones_v[pl.ds(k * 16, 16)] = jnp.full((16,), 1.0, jnp.float32)
    pltpu.sync_copy(ep_hbm, ep_v)

    edge_base = wid * _EPT

    def fire_loads(c, bufs, sem):
        e0 = edge_base + c * _CE
        pltpu.async_copy(src_hbm.at[pl.ds(e0, _CE)], bufs[0], sem)
        pltpu.async_copy(dst_hbm.at[pl.ds(e0, _CE)], bufs[1], sem)
        pltpu.async_copy(len_hbm.at[pl.ds(e0, _CE)], bufs[2], sem)
        pltpu.async_copy(sld_hbm.at[pl.ds(e0, _CE)], bufs[3], sem)

    def wait_loads(c, bufs, sem):
        e0 = edge_base + c * _CE
        pltpu.make_async_copy(src_hbm.at[pl.ds(e0, _CE)], bufs[0], sem).wait()
        pltpu.make_async_copy(dst_hbm.at[pl.ds(e0, _CE)], bufs[1], sem).wait()
        pltpu.make_async_copy(len_hbm.at[pl.ds(e0, _CE)], bufs[2], sem).wait()
        pltpu.make_async_copy(sld_hbm.at[pl.ds(e0, _CE)], bufs[3], sem).wait()

    def compute(bufs):
        src_v, dst_v, lg_v, _ = bufs
        for k in range(_CE // 16):
            sl = pl.ds(k * 16, 16)
            eps = plsc.load_gather(ep_v, [src_v[sl]])
            epd = plsc.load_gather(ep_v, [dst_v[sl]])
            lg_v[sl] = (epd - eps) / lg_v[sl]

    def scatter_descs(bufs, sem):
        src_v, dst_v, lg_v, sld_v = bufs
        return [
            pltpu.make_async_copy(lg_v, acc_g.at[src_v], sem),
            pltpu.make_async_copy(lg_v, acc_g.at[dst_v], sem),
            pltpu.make_async_copy(sld_v, acc_s.at[src_v], sem),
            pltpu.make_async_copy(sld_v, acc_s.at[dst_v], sem),
            pltpu.make_async_copy(ones_v, acc_c.at[src_v], sem),
            pltpu.make_async_copy(ones_v, acc_c.at[dst_v], sem),
        ]

    def fire_scatters(bufs, sem):
        for d in scatter_descs(bufs, sem):
            d.start(add=True)

    def wait_scatters(bufs, sem):
        for d in scatter_descs(bufs, sem):
            d.wait()

    bufs0 = (src0, dst0, lg0, sld0)
    bufs1 = (src1, dst1, lg1, sld1)

    fire_loads(0, bufs0, semL0)
    plsc.subcore_barrier()

    def pair(i, carry):
        c0 = 2 * i
        wait_loads(c0, bufs0, semL0)
        compute(bufs0)

        @pl.when(i > 0)
        def _():
            wait_scatters(bufs1, semS1)

        fire_loads(c0 + 1, bufs1, semL1)
        fire_scatters(bufs0, semS0)
        wait_loads(c0 + 1, bufs1, semL1)
        compute(bufs1)
        wait_scatters(bufs0, semS0)

        @pl.when(i < _FULL // 2 - 1)
        def _():
            fire_loads(c0 + 2, bufs0, semL0)

        fire_scatters(bufs1, semS1)
        return carry

    lax.fori_loop(0, _FULL // 2, pair, 0)
    wait_scatters(bufs1, semS1)

    e0 = edge_base + _FULL * _CE
    pltpu.sync_copy(src_hbm.at[pl.ds(e0, _TAIL)], src0.at[pl.ds(0, _TAIL)])
    pltpu.sync_copy(dst_hbm.at[pl.ds(e0, _TAIL)], dst0.at[pl.ds(0, _TAIL)])
    pltpu.sync_copy(len_hbm.at[pl.ds(e0, _TAIL)], lg0.at[pl.ds(0, _TAIL)])
    pltpu.sync_copy(sld_hbm.at[pl.ds(e0, _TAIL)], sld0.at[pl.ds(0, _TAIL)])
    for k in range(_TAIL // 16, _CE // 16):
        sl = pl.ds(k * 16, 16)
        src0[sl] = jnp.full((16,), _N_NODES, jnp.int32)
        dst0[sl] = jnp.full((16,), _N_NODES, jnp.int32)
        lg0[sl] = jnp.ones((16,), jnp.float32)
        sld0[sl] = jnp.zeros((16,), jnp.float32)
    compute(bufs0)
    fire_scatters(bufs0, semS0)
    wait_scatters(bufs0, semS0)
    plsc.subcore_barrier()

    for a, acc in enumerate((acc_g, acc_s, acc_c)):
        for q in range(7):
            off = sid * _SLICE + q * _QCH
            pltpu.sync_copy(acc.at[pl.ds(off, _QCH)], stg_v)
            pltpu.sync_copy(
                stg_v, out_hbm.at[pl.ds((cid * 3 + a) * _NPAD + off, _QCH)])


def _combine_body(p_ref, ep_ref, geo_ref, dis_ref, out_ref):
    opening_coeff = 1.3455e-09
    closure_coeff = 7.11e-24
    gsum = p_ref[0, 0] + p_ref[1, 0]
    ssum = p_ref[0, 1] + p_ref[1, 1]
    cnt = jnp.maximum(p_ref[0, 2] + p_ref[1, 2], 1.0)
    ep = ep_ref[...]
    dis = dis_ref[...]
    gradient = gsum / cnt + geo_ref[...]
    cavity = jnp.abs(ssum / cnt / _SEC_PER_A) * 0.03
    conduit = (opening_coeff * dis * gradient + cavity) / (
        cavity / 5.74 + closure_coeff * ep * ep * ep)
    conduit = jnp.where(conduit < 1e-06, 1e-06, conduit)
    c54 = conduit * jnp.sqrt(jnp.sqrt(conduit))
    out_ref[...] = dis - (opening_coeff * c54
                          * lax.rsqrt(jnp.abs(gradient) + 1e-12) * gradient)


def kernel(effective_pressure, edge_index, link_length, geometric_gradient,
           discharge, sliding_velocity_link, overburden_pressure, status_at_node):
    f32 = jnp.float32
    npad = _NPAD - _N_NODES
    eff = jnp.pad(effective_pressure, (0, npad), constant_values=1.0)
    over = jnp.pad(overburden_pressure, (0, npad), constant_values=1.0)
    stat = jnp.pad(status_at_node, (0, npad))
    geo = jnp.pad(geometric_gradient, (0, npad))
    dis = jnp.pad(discharge, (0, npad))
    src = edge_index[0]
    dst = edge_index[1]

    ep = pl.pallas_call(
        _ep_body,
        out_shape=jax.ShapeDtypeStruct((784, 128), f32),
    )(eff.reshape(784, 128), over.reshape(784, 128), stat.reshape(784, 128))

    sc = pl.kernel(
        _sc_body,
        out_type=jax.ShapeDtypeStruct((6 * _NPAD,), f32),
        mesh=plsc.VectorSubcoreMesh(core_axis_name="c", subcore_axis_name="s"),
        compiler_params=pltpu.CompilerParams(needs_layout_passes=False),
        scratch_types=[
            pltpu.VMEM((_NPAD,), f32),
            pltpu.VMEM((_CE,), jnp.int32),
            pltpu.VMEM((_CE,), jnp.int32),
            pltpu.VMEM((_CE,), f32),
            pltpu.VMEM((_CE,), f32),
            pltpu.VMEM((_CE,), jnp.int32),
            pltpu.VMEM((_CE,), jnp.int32),
            pltpu.VMEM((_CE,), f32),
            pltpu.VMEM((_CE,), f32),
            pltpu.VMEM((_CE,), f32),
            pltpu.VMEM((_QCH,), f32),
            pltpu.VMEM_SHARED((_NPAD,), f32),
            pltpu.VMEM_SHARED((_NPAD,), f32),
            pltpu.VMEM_SHARED((_NPAD,), f32),
            pltpu.SemaphoreType.DMA,
            pltpu.SemaphoreType.DMA,
            pltpu.SemaphoreType.DMA,
            pltpu.SemaphoreType.DMA,
        ],
    )
    partials = sc(ep.reshape(_NPAD), src, dst, link_length,
                  sliding_velocity_link)

    residual = pl.pallas_call(
        _combine_body,
        out_shape=jax.ShapeDtypeStruct((784, 128), f32),
    )(partials.reshape(2, 3, 784, 128), ep,
      geo.reshape(784, 128), dis.reshape(784, 128))
    return residual.reshape(_NPAD)[:_N_NODES]


# speedup vs baseline: 153.9895x; 1.0182x over previous
import jax
import jax.numpy as jnp
from jax import lax
from jax.experimental import pallas as pl
from jax.experimental.pallas import tpu as pltpu
from jax.experimental.pallas import tpu_sc as plsc

_N_NODES = 100000
_NPAD = 100352
_N_EDGES = 1600000
_EPT = _N_EDGES // 32
_CE = 1024
_FULL = _EPT // _CE
_TAIL = _EPT - _FULL * _CE
_SLICE = _NPAD // 16
_QCH = _SLICE // 7
_SEC_PER_A = 31556926.0


def _sc_body(ep_hbm, src_hbm, dst_hbm, len_hbm, sld_hbm, out_hbm,
             ep_v, src0, dst0, lg0, sld0, src1, dst1, lg1, sld1,
             ones_v, stg_v, acc_g, acc_s, acc_c,
             semL0, semL1, semS0, semS1):
    cid = lax.axis_index("c")
    sid = lax.axis_index("s")
    wid = cid * 16 + sid

    for k in range(_QCH // 16):
        stg_v[pl.ds(k * 16, 16)] = jnp.zeros((16,), jnp.float32)
    for acc in (acc_g, acc_s, acc_c):
        for q in range(7):
            pltpu.sync_copy(stg_v, acc.at[pl.ds(sid * _SLICE + q * _QCH, _QCH)])
    for k in range(_CE // 16):
        ones_v[pl.ds(k * 16, 16)] = jnp.full((16,), 1.0, jnp.float32)
    pltpu.sync_copy(ep_hbm, ep_v.at[pl.ds(0, _N_NODES)])
    ep_v[pl.ds(_N_NODES, 16)] = jnp.ones((16,), jnp.float32)

    edge_base = wid * _EPT

    def fire_loads(c, bufs, sem):
        e0 = edge_base + c * _CE
        pltpu.async_copy(src_hbm.at[pl.ds(e0, _CE)], bufs[0], sem)
        pltpu.async_copy(dst_hbm.at[pl.ds(e0, _CE)], bufs[1], sem)
        pltpu.async_copy(len_hbm.at[pl.ds(e0, _CE)], bufs[2], sem)
        pltpu.async_copy(sld_hbm.at[pl.ds(e0, _CE)], bufs[3], sem)

    def wait_loads(c, bufs, sem):
        e0 = edge_base + c * _CE
        pltpu.make_async_copy(src_hbm.at[pl.ds(e0, _CE)], bufs[0], sem).wait()
        pltpu.make_async_copy(dst_hbm.at[pl.ds(e0, _CE)], bufs[1], sem).wait()
        pltpu.make_async_copy(len_hbm.at[pl.ds(e0, _CE)], bufs[2], sem).wait()
        pltpu.make_async_copy(sld_hbm.at[pl.ds(e0, _CE)], bufs[3], sem).wait()

    def compute(bufs):
        src_v, dst_v, lg_v, _ = bufs
        for k in range(_CE // 16):
            sl = pl.ds(k * 16, 16)
            eps = plsc.load_gather(ep_v, [src_v[sl]])
            epd = plsc.load_gather(ep_v, [dst_v[sl]])
            lg_v[sl] = (epd - eps) / lg_v[sl]

    def scatter_descs(bufs, sem):
        src_v, dst_v, lg_v, sld_v = bufs
        return [
            pltpu.make_async_copy(lg_v, acc_g.at[src_v], sem),
            pltpu.make_async_copy(lg_v, acc_g.at[dst_v], sem),
            pltpu.make_async_copy(sld_v, acc_s.at[src_v], sem),
            pltpu.make_async_copy(sld_v, acc_s.at[dst_v], sem),
            pltpu.make_async_copy(ones_v, acc_c.at[src_v], sem),
            pltpu.make_async_copy(ones_v, acc_c.at[dst_v], sem),
        ]

    def fire_scatters(bufs, sem):
        for d in scatter_descs(bufs, sem):
            d.start(add=True)

    def wait_scatters(bufs, sem):
        for d in scatter_descs(bufs, sem):
            d.wait()

    bufs0 = (src0, dst0, lg0, sld0)
    bufs1 = (src1, dst1, lg1, sld1)

    fire_loads(0, bufs0, semL0)
    plsc.subcore_barrier()

    def pair(i, carry):
        c0 = 2 * i
        wait_loads(c0, bufs0, semL0)
        compute(bufs0)

        @pl.when(i > 0)
        def _():
            wait_scatters(bufs1, semS1)

        fire_loads(c0 + 1, bufs1, semL1)
        fire_scatters(bufs0, semS0)
        wait_loads(c0 + 1, bufs1, semL1)
        compute(bufs1)
        wait_scatters(bufs0, semS0)

        @pl.when(i < _FULL // 2 - 1)
        def _():
            fire_loads(c0 + 2, bufs0, semL0)

        fire_scatters(bufs1, semS1)
        return carry

    lax.fori_loop(0, _FULL // 2, pair, 0)
    wait_scatters(bufs1, semS1)

    e0 = edge_base + _FULL * _CE
    pltpu.sync_copy(src_hbm.at[pl.ds(e0, _TAIL)], src0.at[pl.ds(0, _TAIL)])
    pltpu.sync_copy(dst_hbm.at[pl.ds(e0, _TAIL)], dst0.at[pl.ds(0, _TAIL)])
    pltpu.sync_copy(len_hbm.at[pl.ds(e0, _TAIL)], lg0.at[pl.ds(0, _TAIL)])
    pltpu.sync_copy(sld_hbm.at[pl.ds(e0, _TAIL)], sld0.at[pl.ds(0, _TAIL)])
    for k in range(_TAIL // 16, _CE // 16):
        sl = pl.ds(k * 16, 16)
        src0[sl] = jnp.full((16,), _N_NODES, jnp.int32)
        dst0[sl] = jnp.full((16,), _N_NODES, jnp.int32)
        lg0[sl] = jnp.ones((16,), jnp.float32)
        sld0[sl] = jnp.zeros((16,), jnp.float32)
    compute(bufs0)
    fire_scatters(bufs0, semS0)
    wait_scatters(bufs0, semS0)
    plsc.subcore_barrier()

    for a, acc in enumerate((acc_g, acc_s, acc_c)):
        for q in range(7):
            off = sid * _SLICE + q * _QCH
            pltpu.sync_copy(acc.at[pl.ds(off, _QCH)], stg_v)
            pltpu.sync_copy(
                stg_v, out_hbm.at[pl.ds((cid * 3 + a) * _NPAD + off, _QCH)])


def _combine_body(p_ref, ep_ref, geo_ref, dis_ref, out_ref):
    opening_coeff = 1.3455e-09
    closure_coeff = 7.11e-24
    n = _N_NODES
    gsum = p_ref[pl.ds(0, n)] + p_ref[pl.ds(3 * _NPAD, n)]
    ssum = p_ref[pl.ds(_NPAD, n)] + p_ref[pl.ds(4 * _NPAD, n)]
    cnt = jnp.maximum(p_ref[pl.ds(2 * _NPAD, n)] + p_ref[pl.ds(5 * _NPAD, n)],
                      1.0)
    ep = ep_ref[...]
    dis = dis_ref[...]
    gradient = gsum / cnt + geo_ref[...]
    cavity = jnp.abs(ssum / cnt / _SEC_PER_A) * 0.03
    conduit = (opening_coeff * dis * gradient + cavity) / (
        cavity / 5.74 + closure_coeff * ep * ep * ep)
    conduit = jnp.where(conduit < 1e-06, 1e-06, conduit)
    c54 = conduit * jnp.sqrt(jnp.sqrt(conduit))
    out_ref[...] = dis - (opening_coeff * c54
                          * lax.rsqrt(jnp.abs(gradient) + 1e-12) * gradient)


def kernel(effective_pressure, edge_index, link_length, geometric_gradient,
           discharge, sliding_velocity_link, overburden_pressure, status_at_node):
    f32 = jnp.float32
    src = edge_index[0]
    dst = edge_index[1]
    ep = jnp.where(status_at_node != 0, overburden_pressure,
                   effective_pressure)

    sc = pl.kernel(
        _sc_body,
        out_type=jax.ShapeDtypeStruct((6 * _NPAD,), f32),
        mesh=plsc.VectorSubcoreMesh(core_axis_name="c", subcore_axis_name="s"),
        compiler_params=pltpu.CompilerParams(needs_layout_passes=False),
        scratch_types=[
            pltpu.VMEM((_N_NODES + 16,), f32),
            pltpu.VMEM((_CE,), jnp.int32),
            pltpu.VMEM((_CE,), jnp.int32),
            pltpu.VMEM((_CE,), f32),
            pltpu.VMEM((_CE,), f32),
            pltpu.VMEM((_CE,), jnp.int32),
            pltpu.VMEM((_CE,), jnp.int32),
            pltpu.VMEM((_CE,), f32),
            pltpu.VMEM((_CE,), f32),
            pltpu.VMEM((_CE,), f32),
            pltpu.VMEM((_QCH,), f32),
            pltpu.VMEM_SHARED((_NPAD,), f32),
            pltpu.VMEM_SHARED((_NPAD,), f32),
            pltpu.VMEM_SHARED((_NPAD,), f32),
            pltpu.SemaphoreType.DMA,
            pltpu.SemaphoreType.DMA,
            pltpu.SemaphoreType.DMA,
            pltpu.SemaphoreType.DMA,
        ],
    )
    partials = sc(ep, src, dst, link_length, sliding_velocity_link)

    residual = pl.pallas_call(
        _combine_body,
        out_shape=jax.ShapeDtypeStruct((_N_NODES,), f32),
    )(partials, ep, geometric_gradient, discharge)
    return residual
